# baseline (speedup 1.0000x reference)
import jax
import jax.numpy as jnp
from jax import lax
from jax.experimental import pallas as pl
from jax.experimental.pallas import tpu as pltpu
from jax.experimental.pallas import tpu_sc as plsc

_E = 8
_I = 512
_T = 2048
_H = 1024
_TB = 256
_RB = 512
_AB = 512
_BM = 512
_N = _T * 2
_NP = _N + _E * _BM
_NB = _NP // _BM
_NW = 32
_CC = _T // _NW


def _silu(x):
    return x * jax.nn.sigmoid(x)


def _dot(a, b, dims):
    return jax.lax.dot_general(a, b, (dims, ((), ())),
                               preferred_element_type=jnp.float32,
                               precision=jax.lax.Precision.DEFAULT)


def _router_body(x_ref, wg_ref, wts_ref, pos1_ref, pos2_ref, bexp_ref,
                 nact_ref, sel_s, wts_s, counts_s, poff_s, carry_s):
    p = pl.program_id(0)
    t = pl.program_id(1)
    hp = jax.lax.Precision.HIGHEST

    @pl.when(p == 0)
    def _pass0():
        x = x_ref[...]
        logits = _dot(x, wg_ref[...], ((1,), (1,)))
        m = jnp.max(logits, axis=1, keepdims=True)
        pr = jnp.exp(logits - m)
        iota = jax.lax.broadcasted_iota(jnp.int32, pr.shape, 1)
        m1 = jnp.max(pr, axis=1, keepdims=True)
        i1 = jnp.min(jnp.where(pr == m1, iota, _E), axis=1, keepdims=True)
        pr2 = jnp.where(iota == i1, -1.0, pr)
        m2 = jnp.max(pr2, axis=1, keepdims=True)
        i2 = jnp.min(jnp.where(pr2 == m2, iota, _E), axis=1, keepdims=True)
        den = m1 + m2 + 1e-20
        w12 = jnp.concatenate([m1 / den, m2 / den], axis=1)
        wts_ref[...] = w12
        sel_s[pl.ds(t * _RB, _RB), :] = jnp.concatenate([i1, i2], axis=1)
        wts_s[pl.ds(t * _RB, _RB), :] = w12

        both = ((iota == i1) | (iota == i2)).astype(jnp.float32)
        bsum = jnp.sum(both, axis=0, keepdims=True)

        @pl.when(t == 0)
        def _():
            counts_s[...] = bsum

        @pl.when(t > 0)
        def _():
            counts_s[...] += bsum

        @pl.when(t == (_T // _RB) - 1)
        def _finalize():
            counts = counts_s[...]
            padded = jnp.floor((counts + (_BM - 1)) / _BM) * _BM
            ir = jax.lax.broadcasted_iota(jnp.int32, (_E, _E), 0)
            ic = jax.lax.broadcasted_iota(jnp.int32, (_E, _E), 1)
            tri = (ir <= ic).astype(jnp.float32)
            cpad = jax.lax.dot_general(padded, tri, (((1,), (0,)), ((), ())),
                                       preferred_element_type=jnp.float32,
                                       precision=hp)
            poff_s[...] = cpad - padded
            nact_ref[...] = (cpad[:, _E - 1:] / _BM).astype(jnp.int32)
            cpadB = jnp.broadcast_to(cpad, (_NB, _E))
            bsB = (jax.lax.broadcasted_iota(jnp.int32, (_NB, _E), 0)
                   .astype(jnp.float32) * _BM)
            be = jnp.sum((cpadB <= bsB).astype(jnp.int32), axis=1)
            bexp_ref[...] = jnp.minimum(be, _E - 1)[None, :]

    @pl.when(p == 1)
    def _pass1():
        sel = sel_s[pl.ds(t * _RB, _RB), :]
        iota = jax.lax.broadcasted_iota(jnp.int32, (_RB, _E), 1)
        oh1 = (iota == sel[:, 0][:, None]).astype(jnp.float32)
        oh2 = (iota == sel[:, 1][:, None]).astype(jnp.float32)
        both = oh1 + oh2

        @pl.when(t == 0)
        def _():
            carry_s[...] = jnp.zeros_like(carry_s)

        ir = jax.lax.broadcasted_iota(jnp.int32, (_RB, _RB), 0)
        ic = jax.lax.broadcasted_iota(jnp.int32, (_RB, _RB), 1)
        lstrict = (ir > ic).astype(jnp.float32)
        cume = jax.lax.dot_general(lstrict, both, (((1,), (0,)), ((), ())),
                                   preferred_element_type=jnp.float32,
                                   precision=hp)
        base = poff_s[...] + carry_s[...] + cume
        pos1_ref[...] = jnp.sum(oh1 * base, axis=1).astype(jnp.int32)
        pos2_ref[...] = jnp.sum(oh2 * base, axis=1).astype(jnp.int32)
        carry_s[...] += jnp.sum(both, axis=0, keepdims=True)
        wts_ref[...] = wts_s[pl.ds(t * _RB, _RB), :]


def _shared_body(x_ref, sgw_ref, suw_ref, sdw_ref, out_ref):
    x = x_ref[...]
    g = _dot(x, sgw_ref[...], ((1,), (1,)))
    u = _dot(x, suw_ref[...], ((1,), (1,)))
    out_ref[...] = _dot(_silu(g) * u, sdw_ref[...], ((1,), (1,)))


def _gmm_body(bexp_ref, nact_ref, xs_ref, gw_ref, uw_ref, dw_ref, h_ref):
    b = pl.program_id(0)
    ih = pl.program_id(1)

    @pl.when(b < nact_ref[0, 0])
    def _compute():
        x = xs_ref[...]
        g = _dot(x, gw_ref[0], ((1,), (1,)))
        u = _dot(x, uw_ref[0], ((1,), (1,)))
        a = _silu(g) * u
        part = _dot(a, dw_ref[0], ((1,), (1,)))

        @pl.when(ih == 0)
        def _():
            h_ref[...] = part

        @pl.when(ih == 1)
        def _():
            h_ref[...] += part

    @pl.when(jnp.logical_and(b >= nact_ref[0, 0], ih == 0))
    def _skip():
        h_ref[...] = jnp.zeros_like(h_ref)


def _sc_dispatch_body(x_hbm, pos1_hbm, pos2_hbm, xs_hbm,
                      p1_v, p2_v, rows_v, sem1, sem2):
    wid = lax.axis_index("s") * 2 + lax.axis_index("c")
    tb = wid * _CC
    pltpu.sync_copy(pos1_hbm.at[pl.ds(tb, _CC)], p1_v)
    pltpu.sync_copy(pos2_hbm.at[pl.ds(tb, _CC)], p2_v)
    pltpu.sync_copy(x_hbm.at[pl.ds(tb, _CC)], rows_v)
    c1 = pltpu.async_copy(rows_v, xs_hbm.at[p1_v], sem1)
    c2 = pltpu.async_copy(rows_v, xs_hbm.at[p2_v], sem2)
    c1.wait()
    c2.wait()


def _sc_unsort_body(h_hbm, pos1_hbm, pos2_hbm, h1_hbm, h2_hbm,
                    idx_v, rows_v, sem):
    wid = lax.axis_index("s") * 2 + lax.axis_index("c")
    gb = wid * _CC
    for pos_hbm, dst_hbm in ((pos1_hbm, h1_hbm), (pos2_hbm, h2_hbm)):
        pltpu.sync_copy(pos_hbm.at[pl.ds(gb, _CC)], idx_v)
        pltpu.async_copy(h_hbm.at[idx_v], rows_v, sem).wait()
        pltpu.sync_copy(rows_v, dst_hbm.at[pl.ds(gb, _CC)])


def _add_body(s_ref, h1_ref, h2_ref, w_ref, out_ref):
    w = w_ref[...]
    out_ref[...] = (s_ref[...] + w[:, 0][:, None] * h1_ref[...]
                    + w[:, 1][:, None] * h2_ref[...])


def kernel(hidden_states, W_gate, gate_w, up_w, down_w,
           shared_gate_w, shared_up_w, shared_down_w):
    bsz, seq, hdim = hidden_states.shape
    x = hidden_states.reshape(_T, _H)
    f32 = jnp.float32
    i32 = jnp.int32

    wts, pos1, pos2, bexp, nact = pl.pallas_call(
        _router_body,
        grid=(2, _T // _RB),
        in_specs=[
            pl.BlockSpec((_RB, _H), lambda p, t: (t * (1 - p), 0)),
            pl.BlockSpec((_E, _H), lambda p, t: (0, 0)),
        ],
        out_specs=[
            pl.BlockSpec((_RB, 2), lambda p, t: (t, 0)),
            pl.BlockSpec((_RB,), lambda p, t: (t,)),
            pl.BlockSpec((_RB,), lambda p, t: (t,)),
            pl.BlockSpec((1, _NB), lambda p, t: (0, 0)),
            pl.BlockSpec((1, 1), lambda p, t: (0, 0)),
        ],
        out_shape=[
            jax.ShapeDtypeStruct((_T, 2), f32),
            jax.ShapeDtypeStruct((_T,), i32),
            jax.ShapeDtypeStruct((_T,), i32),
            jax.ShapeDtypeStruct((1, _NB), i32),
            jax.ShapeDtypeStruct((1, 1), i32),
        ],
        scratch_shapes=[
            pltpu.VMEM((_T, 2), i32),
            pltpu.VMEM((_T, 2), f32),
            pltpu.VMEM((1, _E), f32),
            pltpu.VMEM((1, _E), f32),
            pltpu.VMEM((1, _E), f32),
        ],
        compiler_params=pltpu.CompilerParams(
            dimension_semantics=("arbitrary", "arbitrary")),
    )(x, W_gate)

    mesh = plsc.VectorSubcoreMesh(core_axis_name="c", subcore_axis_name="s")

    xs = pl.kernel(
        _sc_dispatch_body, mesh=mesh,
        out_type=jax.ShapeDtypeStruct((_NP, _H), f32),
        scratch_types=[
            pltpu.VMEM((_CC,), i32),
            pltpu.VMEM((_CC,), i32),
            pltpu.VMEM((_CC, _H), f32),
            pltpu.SemaphoreType.DMA,
            pltpu.SemaphoreType.DMA,
        ],
    )(x, pos1, pos2)

    shared_out = pl.pallas_call(
        _shared_body,
        grid=(_T // _TB,),
        in_specs=[
            pl.BlockSpec((_TB, _H), lambda t: (t, 0)),
            pl.BlockSpec(shared_gate_w.shape, lambda t: (0, 0)),
            pl.BlockSpec(shared_up_w.shape, lambda t: (0, 0)),
            pl.BlockSpec(shared_down_w.shape, lambda t: (0, 0)),
        ],
        out_specs=pl.BlockSpec((_TB, _H), lambda t: (t, 0)),
        out_shape=jax.ShapeDtypeStruct((_T, _H), f32),
        compiler_params=pltpu.CompilerParams(
            dimension_semantics=("arbitrary",)),
    )(x, shared_gate_w, shared_up_w, shared_down_w)

    hi = _I // 2
    grid_spec = pltpu.PrefetchScalarGridSpec(
        num_scalar_prefetch=2,
        grid=(_NB, 2),
        in_specs=[
            pl.BlockSpec((_BM, _H), lambda b, ih, be, na: (b, 0)),
            pl.BlockSpec((1, hi, _H), lambda b, ih, be, na: (be[0, b], ih, 0)),
            pl.BlockSpec((1, hi, _H), lambda b, ih, be, na: (be[0, b], ih, 0)),
            pl.BlockSpec((1, _H, hi), lambda b, ih, be, na: (be[0, b], 0, ih)),
        ],
        out_specs=pl.BlockSpec((_BM, _H), lambda b, ih, be, na: (b, 0)),
    )
    h_sorted = pl.pallas_call(
        _gmm_body,
        grid_spec=grid_spec,
        out_shape=jax.ShapeDtypeStruct((_NP, _H), f32),
        compiler_params=pltpu.CompilerParams(
            dimension_semantics=("arbitrary", "arbitrary")),
    )(bexp, nact, xs, gate_w, up_w, down_w)

    h1, h2 = pl.kernel(
        _sc_unsort_body, mesh=mesh,
        out_type=[
            jax.ShapeDtypeStruct((_T, _H), f32),
            jax.ShapeDtypeStruct((_T, _H), f32),
        ],
        scratch_types=[
            pltpu.VMEM((_CC,), i32),
            pltpu.VMEM((_CC, _H), f32),
            pltpu.SemaphoreType.DMA,
        ],
    )(h_sorted, pos1, pos2)

    out = pl.pallas_call(
        _add_body,
        grid=(_T // _AB,),
        in_specs=[pl.BlockSpec((_AB, _H), lambda t: (t, 0))] * 3
        + [pl.BlockSpec((_AB, 2), lambda t: (t, 0))],
        out_specs=pl.BlockSpec((_AB, _H), lambda t: (t, 0)),
        out_shape=jax.ShapeDtypeStruct((_T, _H), f32),
        compiler_params=pltpu.CompilerParams(
            dimension_semantics=("arbitrary",)),
    )(shared_out, h1, h2, wts)

    return out.reshape(bsz, seq, hdim)


# speedup vs baseline: 1.1520x; 1.1520x over previous
import jax
import jax.numpy as jnp
from jax import lax
from jax.experimental import pallas as pl
from jax.experimental.pallas import tpu as pltpu
from jax.experimental.pallas import tpu_sc as plsc

_E = 8
_I = 512
_T = 2048
_H = 1024
_TB = 256
_RB = 512
_AB = 512
_BM = 512
_N = _T * 2
_NP = _N + _E * _BM
_NB = _NP // _BM
_NW = 32
_CC = _T // _NW


def _silu(x):
    return x * jax.nn.sigmoid(x)


def _dot(a, b, dims):
    return jax.lax.dot_general(a, b, (dims, ((), ())),
                               preferred_element_type=jnp.float32,
                               precision=jax.lax.Precision.DEFAULT)


def _router_body(x_ref, wg_ref, wts_ref, pos1_ref, pos2_ref, bexp_ref,
                 nact_ref, sel_s, wts_s, counts_s, poff_s, carry_s):
    p = pl.program_id(0)
    t = pl.program_id(1)
    hp = jax.lax.Precision.HIGHEST

    @pl.when(p == 0)
    def _pass0():
        x = x_ref[...]
        logits = _dot(x, wg_ref[...], ((1,), (1,)))
        m = jnp.max(logits, axis=1, keepdims=True)
        pr = jnp.exp(logits - m)
        iota = jax.lax.broadcasted_iota(jnp.int32, pr.shape, 1)
        m1 = jnp.max(pr, axis=1, keepdims=True)
        i1 = jnp.min(jnp.where(pr == m1, iota, _E), axis=1, keepdims=True)
        pr2 = jnp.where(iota == i1, -1.0, pr)
        m2 = jnp.max(pr2, axis=1, keepdims=True)
        i2 = jnp.min(jnp.where(pr2 == m2, iota, _E), axis=1, keepdims=True)
        den = m1 + m2 + 1e-20
        w12 = jnp.concatenate([m1 / den, m2 / den], axis=1)
        wts_ref[...] = w12
        sel_s[pl.ds(t * _RB, _RB), :] = jnp.concatenate([i1, i2], axis=1)
        wts_s[pl.ds(t * _RB, _RB), :] = w12

        both = ((iota == i1) | (iota == i2)).astype(jnp.float32)
        bsum = jnp.sum(both, axis=0, keepdims=True)

        @pl.when(t == 0)
        def _():
            counts_s[...] = bsum

        @pl.when(t > 0)
        def _():
            counts_s[...] += bsum

        @pl.when(t == (_T // _RB) - 1)
        def _finalize():
            counts = counts_s[...]
            padded = jnp.floor((counts + (_BM - 1)) / _BM) * _BM
            ir = jax.lax.broadcasted_iota(jnp.int32, (_E, _E), 0)
            ic = jax.lax.broadcasted_iota(jnp.int32, (_E, _E), 1)
            tri = (ir <= ic).astype(jnp.float32)
            cpad = jax.lax.dot_general(padded, tri, (((1,), (0,)), ((), ())),
                                       preferred_element_type=jnp.float32,
                                       precision=hp)
            poff_s[...] = cpad - padded
            nact_ref[...] = (cpad[:, _E - 1:] / _BM).astype(jnp.int32)
            cpadB = jnp.broadcast_to(cpad, (_NB, _E))
            bsB = (jax.lax.broadcasted_iota(jnp.int32, (_NB, _E), 0)
                   .astype(jnp.float32) * _BM)
            be = jnp.sum((cpadB <= bsB).astype(jnp.int32), axis=1)
            bexp_ref[...] = jnp.minimum(be, _E - 1)[None, :]

    @pl.when(p == 1)
    def _pass1():
        sel = sel_s[pl.ds(t * _RB, _RB), :]
        iota = jax.lax.broadcasted_iota(jnp.int32, (_RB, _E), 1)
        oh1 = (iota == sel[:, 0][:, None]).astype(jnp.float32)
        oh2 = (iota == sel[:, 1][:, None]).astype(jnp.float32)
        both = oh1 + oh2

        @pl.when(t == 0)
        def _():
            carry_s[...] = jnp.zeros_like(carry_s)

        ir = jax.lax.broadcasted_iota(jnp.int32, (_RB, _RB), 0)
        ic = jax.lax.broadcasted_iota(jnp.int32, (_RB, _RB), 1)
        lstrict = (ir > ic).astype(jnp.float32)
        cume = jax.lax.dot_general(lstrict, both, (((1,), (0,)), ((), ())),
                                   preferred_element_type=jnp.float32,
                                   precision=hp)
        base = poff_s[...] + carry_s[...] + cume
        pos1_ref[...] = jnp.sum(oh1 * base, axis=1).astype(jnp.int32)
        pos2_ref[...] = jnp.sum(oh2 * base, axis=1).astype(jnp.int32)
        carry_s[...] += jnp.sum(both, axis=0, keepdims=True)
        wts_ref[...] = wts_s[pl.ds(t * _RB, _RB), :]


def _shared_body(x_ref, sgw_ref, suw_ref, sdw_ref, out_ref):
    x = x_ref[...]
    g = _dot(x, sgw_ref[...], ((1,), (1,)))
    u = _dot(x, suw_ref[...], ((1,), (1,)))
    out_ref[...] = _dot(_silu(g) * u, sdw_ref[...], ((1,), (1,)))


def _gmm_body(bexp_ref, nact_ref, xs_ref, gw_ref, uw_ref, dw_ref, h_ref):
    b = pl.program_id(0)

    @pl.when(b < nact_ref[0, 0])
    def _compute():
        x = xs_ref[...]
        g = _dot(x, gw_ref[0], ((1,), (1,)))
        u = _dot(x, uw_ref[0], ((1,), (1,)))
        a = _silu(g) * u
        h_ref[...] = _dot(a, dw_ref[0], ((1,), (1,)))

    @pl.when(b >= nact_ref[0, 0])
    def _skip():
        h_ref[...] = jnp.zeros_like(h_ref)


def _sc_dispatch_body(x_hbm, pos1_hbm, pos2_hbm, xs_hbm,
                      p1_v, p2_v, rows_v, sem1, sem2):
    wid = lax.axis_index("s") * 2 + lax.axis_index("c")
    tb = wid * _CC
    pltpu.sync_copy(pos1_hbm.at[pl.ds(tb, _CC)], p1_v)
    pltpu.sync_copy(pos2_hbm.at[pl.ds(tb, _CC)], p2_v)
    pltpu.sync_copy(x_hbm.at[pl.ds(tb, _CC)], rows_v)
    c1 = pltpu.async_copy(rows_v, xs_hbm.at[p1_v], sem1)
    c2 = pltpu.async_copy(rows_v, xs_hbm.at[p2_v], sem2)
    c1.wait()
    c2.wait()


def _sc_unsort_body(h_hbm, pos1_hbm, pos2_hbm, h1_hbm, h2_hbm,
                    idx_v, rows_v, sem):
    wid = lax.axis_index("s") * 2 + lax.axis_index("c")
    gb = wid * _CC
    for pos_hbm, dst_hbm in ((pos1_hbm, h1_hbm), (pos2_hbm, h2_hbm)):
        pltpu.sync_copy(pos_hbm.at[pl.ds(gb, _CC)], idx_v)
        pltpu.async_copy(h_hbm.at[idx_v], rows_v, sem).wait()
        pltpu.sync_copy(rows_v, dst_hbm.at[pl.ds(gb, _CC)])


def _add_body(s_ref, h1_ref, h2_ref, w_ref, out_ref):
    w = w_ref[...]
    out_ref[...] = (s_ref[...] + w[:, 0][:, None] * h1_ref[...]
                    + w[:, 1][:, None] * h2_ref[...])


def kernel(hidden_states, W_gate, gate_w, up_w, down_w,
           shared_gate_w, shared_up_w, shared_down_w):
    bsz, seq, hdim = hidden_states.shape
    x = hidden_states.reshape(_T, _H)
    f32 = jnp.float32
    i32 = jnp.int32

    wts, pos1, pos2, bexp, nact = pl.pallas_call(
        _router_body,
        grid=(2, _T // _RB),
        in_specs=[
            pl.BlockSpec((_RB, _H), lambda p, t: (t * (1 - p), 0)),
            pl.BlockSpec((_E, _H), lambda p, t: (0, 0)),
        ],
        out_specs=[
            pl.BlockSpec((_RB, 2), lambda p, t: (t, 0)),
            pl.BlockSpec((_RB,), lambda p, t: (t,)),
            pl.BlockSpec((_RB,), lambda p, t: (t,)),
            pl.BlockSpec((1, _NB), lambda p, t: (0, 0)),
            pl.BlockSpec((1, 1), lambda p, t: (0, 0)),
        ],
        out_shape=[
            jax.ShapeDtypeStruct((_T, 2), f32),
            jax.ShapeDtypeStruct((_T,), i32),
            jax.ShapeDtypeStruct((_T,), i32),
            jax.ShapeDtypeStruct((1, _NB), i32),
            jax.ShapeDtypeStruct((1, 1), i32),
        ],
        scratch_shapes=[
            pltpu.VMEM((_T, 2), i32),
            pltpu.VMEM((_T, 2), f32),
            pltpu.VMEM((1, _E), f32),
            pltpu.VMEM((1, _E), f32),
            pltpu.VMEM((1, _E), f32),
        ],
        compiler_params=pltpu.CompilerParams(
            dimension_semantics=("arbitrary", "arbitrary")),
    )(x, W_gate)

    mesh = plsc.VectorSubcoreMesh(core_axis_name="c", subcore_axis_name="s")

    xs = pl.kernel(
        _sc_dispatch_body, mesh=mesh,
        out_type=jax.ShapeDtypeStruct((_NP, _H), f32),
        scratch_types=[
            pltpu.VMEM((_CC,), i32),
            pltpu.VMEM((_CC,), i32),
            pltpu.VMEM((_CC, _H), f32),
            pltpu.SemaphoreType.DMA,
            pltpu.SemaphoreType.DMA,
        ],
    )(x, pos1, pos2)

    shared_out = pl.pallas_call(
        _shared_body,
        grid=(_T // _TB,),
        in_specs=[
            pl.BlockSpec((_TB, _H), lambda t: (t, 0)),
            pl.BlockSpec(shared_gate_w.shape, lambda t: (0, 0)),
            pl.BlockSpec(shared_up_w.shape, lambda t: (0, 0)),
            pl.BlockSpec(shared_down_w.shape, lambda t: (0, 0)),
        ],
        out_specs=pl.BlockSpec((_TB, _H), lambda t: (t, 0)),
        out_shape=jax.ShapeDtypeStruct((_T, _H), f32),
        compiler_params=pltpu.CompilerParams(
            dimension_semantics=("arbitrary",)),
    )(x, shared_gate_w, shared_up_w, shared_down_w)

    grid_spec = pltpu.PrefetchScalarGridSpec(
        num_scalar_prefetch=2,
        grid=(_NB,),
        in_specs=[
            pl.BlockSpec((_BM, _H), lambda b, be, na: (b, 0)),
            pl.BlockSpec((1, _I, _H), lambda b, be, na: (be[0, b], 0, 0)),
            pl.BlockSpec((1, _I, _H), lambda b, be, na: (be[0, b], 0, 0)),
            pl.BlockSpec((1, _H, _I), lambda b, be, na: (be[0, b], 0, 0)),
        ],
        out_specs=pl.BlockSpec((_BM, _H), lambda b, be, na: (b, 0)),
    )
    h_sorted = pl.pallas_call(
        _gmm_body,
        grid_spec=grid_spec,
        out_shape=jax.ShapeDtypeStruct((_NP, _H), f32),
        compiler_params=pltpu.CompilerParams(
            dimension_semantics=("arbitrary",)),
    )(bexp, nact, xs, gate_w, up_w, down_w)

    h1, h2 = pl.kernel(
        _sc_unsort_body, mesh=mesh,
        out_type=[
            jax.ShapeDtypeStruct((_T, _H), f32),
            jax.ShapeDtypeStruct((_T, _H), f32),
        ],
        scratch_types=[
            pltpu.VMEM((_CC,), i32),
            pltpu.VMEM((_CC, _H), f32),
            pltpu.SemaphoreType.DMA,
        ],
    )(h_sorted, pos1, pos2)

    out = pl.pallas_call(
        _add_body,
        grid=(_T // _AB,),
        in_specs=[pl.BlockSpec((_AB, _H), lambda t: (t, 0))] * 3
        + [pl.BlockSpec((_AB, 2), lambda t: (t, 0))],
        out_specs=pl.BlockSpec((_AB, _H), lambda t: (t, 0)),
        out_shape=jax.ShapeDtypeStruct((_T, _H), f32),
        compiler_params=pltpu.CompilerParams(
            dimension_semantics=("arbitrary",)),
    )(shared_out, h1, h2, wts)

    return out.reshape(bsz, seq, hdim)


# speedup vs baseline: 1.1686x; 1.0144x over previous
import jax
import jax.numpy as jnp
from jax import lax
from jax.experimental import pallas as pl
from jax.experimental.pallas import tpu as pltpu
from jax.experimental.pallas import tpu_sc as plsc

_E = 8
_I = 512
_T = 2048
_H = 1024
_TB = 512
_RB = 512
_AB = 1024
_BM = 512
_N = _T * 2
_NP = _N + _E * _BM
_NB = _NP // _BM
_NW = 32
_CC = _T // _NW


def _silu(x):
    return x * jax.nn.sigmoid(x)


def _dot(a, b, dims):
    return jax.lax.dot_general(a, b, (dims, ((), ())),
                               preferred_element_type=jnp.float32,
                               precision=jax.lax.Precision.DEFAULT)


def _router_body(x_ref, wg_ref, wts_ref, pos1_ref, pos2_ref, bexp_ref,
                 nact_ref, sel_s, wts_s, counts_s, poff_s, carry_s):
    p = pl.program_id(0)
    t = pl.program_id(1)
    hp = jax.lax.Precision.HIGHEST

    @pl.when(p == 0)
    def _pass0():
        x = x_ref[...]
        logits = _dot(x, wg_ref[...], ((1,), (1,)))
        m = jnp.max(logits, axis=1, keepdims=True)
        pr = jnp.exp(logits - m)
        iota = jax.lax.broadcasted_iota(jnp.int32, pr.shape, 1)
        m1 = jnp.max(pr, axis=1, keepdims=True)
        i1 = jnp.min(jnp.where(pr == m1, iota, _E), axis=1, keepdims=True)
        pr2 = jnp.where(iota == i1, -1.0, pr)
        m2 = jnp.max(pr2, axis=1, keepdims=True)
        i2 = jnp.min(jnp.where(pr2 == m2, iota, _E), axis=1, keepdims=True)
        den = m1 + m2 + 1e-20
        w12 = jnp.concatenate([m1 / den, m2 / den], axis=1)
        wts_ref[...] = w12
        sel_s[pl.ds(t * _RB, _RB), :] = jnp.concatenate([i1, i2], axis=1)
        wts_s[pl.ds(t * _RB, _RB), :] = w12

        both = ((iota == i1) | (iota == i2)).astype(jnp.float32)
        bsum = jnp.sum(both, axis=0, keepdims=True)

        @pl.when(t == 0)
        def _():
            counts_s[...] = bsum

        @pl.when(t > 0)
        def _():
            counts_s[...] += bsum

        @pl.when(t == (_T // _RB) - 1)
        def _finalize():
            counts = counts_s[...]
            padded = jnp.floor((counts + (_BM - 1)) / _BM) * _BM
            ir = jax.lax.broadcasted_iota(jnp.int32, (_E, _E), 0)
            ic = jax.lax.broadcasted_iota(jnp.int32, (_E, _E), 1)
            tri = (ir <= ic).astype(jnp.float32)
            cpad = jax.lax.dot_general(padded, tri, (((1,), (0,)), ((), ())),
                                       preferred_element_type=jnp.float32,
                                       precision=hp)
            poff_s[...] = cpad - padded
            nact_ref[...] = (cpad[:, _E - 1:] / _BM).astype(jnp.int32)
            cpadB = jnp.broadcast_to(cpad, (_NB, _E))
            bsB = (jax.lax.broadcasted_iota(jnp.int32, (_NB, _E), 0)
                   .astype(jnp.float32) * _BM)
            be = jnp.sum((cpadB <= bsB).astype(jnp.int32), axis=1)
            bexp_ref[...] = jnp.minimum(be, _E - 1)[None, :]

    @pl.when(p == 1)
    def _pass1():
        sel = sel_s[pl.ds(t * _RB, _RB), :]
        iota = jax.lax.broadcasted_iota(jnp.int32, (_RB, _E), 1)
        oh1 = (iota == sel[:, 0][:, None]).astype(jnp.float32)
        oh2 = (iota == sel[:, 1][:, None]).astype(jnp.float32)
        both = oh1 + oh2

        @pl.when(t == 0)
        def _():
            carry_s[...] = jnp.zeros_like(carry_s)

        ir = jax.lax.broadcasted_iota(jnp.int32, (_RB, _RB), 0)
        ic = jax.lax.broadcasted_iota(jnp.int32, (_RB, _RB), 1)
        lstrict = (ir > ic).astype(jnp.float32)
        cume = jax.lax.dot_general(lstrict, both, (((1,), (0,)), ((), ())),
                                   preferred_element_type=jnp.float32,
                                   precision=hp)
        base = poff_s[...] + carry_s[...] + cume
        pos1_ref[...] = jnp.sum(oh1 * base, axis=1).astype(jnp.int32)
        pos2_ref[...] = jnp.sum(oh2 * base, axis=1).astype(jnp.int32)
        carry_s[...] += jnp.sum(both, axis=0, keepdims=True)
        wts_ref[...] = wts_s[pl.ds(t * _RB, _RB), :]


def _shared_body(x_ref, sgw_ref, suw_ref, sdw_ref, out_ref):
    x = x_ref[...]
    g = _dot(x, sgw_ref[...], ((1,), (1,)))
    u = _dot(x, suw_ref[...], ((1,), (1,)))
    out_ref[...] = _dot(_silu(g) * u, sdw_ref[...], ((1,), (1,)))


def _gmm_body(bexp_ref, nact_ref, xs_ref, gw_ref, uw_ref, dw_ref, h_ref):
    b = pl.program_id(0)

    @pl.when(b < nact_ref[0, 0])
    def _compute():
        x = xs_ref[...]
        g = _dot(x, gw_ref[0], ((1,), (1,)))
        u = _dot(x, uw_ref[0], ((1,), (1,)))
        a = _silu(g) * u
        h_ref[...] = _dot(a, dw_ref[0], ((1,), (1,)))

    @pl.when(b >= nact_ref[0, 0])
    def _skip():
        h_ref[...] = jnp.zeros_like(h_ref)


def _sc_dispatch_body(x_hbm, pos1_hbm, pos2_hbm, xs_hbm,
                      p1_v, p2_v, rows_v, sem1, sem2):
    wid = lax.axis_index("s") * 2 + lax.axis_index("c")
    tb = wid * _CC
    pltpu.sync_copy(pos1_hbm.at[pl.ds(tb, _CC)], p1_v)
    pltpu.sync_copy(pos2_hbm.at[pl.ds(tb, _CC)], p2_v)
    pltpu.sync_copy(x_hbm.at[pl.ds(tb, _CC)], rows_v)
    c1 = pltpu.async_copy(rows_v, xs_hbm.at[p1_v], sem1)
    c2 = pltpu.async_copy(rows_v, xs_hbm.at[p2_v], sem2)
    c1.wait()
    c2.wait()


def _sc_unsort_body(h_hbm, pos1_hbm, pos2_hbm, h1_hbm, h2_hbm,
                    idx_v, rows_v, sem):
    wid = lax.axis_index("s") * 2 + lax.axis_index("c")
    gb = wid * _CC
    for pos_hbm, dst_hbm in ((pos1_hbm, h1_hbm), (pos2_hbm, h2_hbm)):
        pltpu.sync_copy(pos_hbm.at[pl.ds(gb, _CC)], idx_v)
        pltpu.async_copy(h_hbm.at[idx_v], rows_v, sem).wait()
        pltpu.sync_copy(rows_v, dst_hbm.at[pl.ds(gb, _CC)])


def _add_body(s_ref, h1_ref, h2_ref, w_ref, out_ref):
    w = w_ref[...]
    out_ref[...] = (s_ref[...] + w[:, 0][:, None] * h1_ref[...]
                    + w[:, 1][:, None] * h2_ref[...])


def kernel(hidden_states, W_gate, gate_w, up_w, down_w,
           shared_gate_w, shared_up_w, shared_down_w):
    bsz, seq, hdim = hidden_states.shape
    x = hidden_states.reshape(_T, _H)
    f32 = jnp.float32
    i32 = jnp.int32

    wts, pos1, pos2, bexp, nact = pl.pallas_call(
        _router_body,
        grid=(2, _T // _RB),
        in_specs=[
            pl.BlockSpec((_RB, _H), lambda p, t: (t * (1 - p), 0)),
            pl.BlockSpec((_E, _H), lambda p, t: (0, 0)),
        ],
        out_specs=[
            pl.BlockSpec((_RB, 2), lambda p, t: (t, 0)),
            pl.BlockSpec((_RB,), lambda p, t: (t,)),
            pl.BlockSpec((_RB,), lambda p, t: (t,)),
            pl.BlockSpec((1, _NB), lambda p, t: (0, 0)),
            pl.BlockSpec((1, 1), lambda p, t: (0, 0)),
        ],
        out_shape=[
            jax.ShapeDtypeStruct((_T, 2), f32),
            jax.ShapeDtypeStruct((_T,), i32),
            jax.ShapeDtypeStruct((_T,), i32),
            jax.ShapeDtypeStruct((1, _NB), i32),
            jax.ShapeDtypeStruct((1, 1), i32),
        ],
        scratch_shapes=[
            pltpu.VMEM((_T, 2), i32),
            pltpu.VMEM((_T, 2), f32),
            pltpu.VMEM((1, _E), f32),
            pltpu.VMEM((1, _E), f32),
            pltpu.VMEM((1, _E), f32),
        ],
        compiler_params=pltpu.CompilerParams(
            dimension_semantics=("arbitrary", "arbitrary")),
    )(x, W_gate)

    mesh = plsc.VectorSubcoreMesh(core_axis_name="c", subcore_axis_name="s")

    xs = pl.kernel(
        _sc_dispatch_body, mesh=mesh,
        out_type=jax.ShapeDtypeStruct((_NP, _H), f32),
        scratch_types=[
            pltpu.VMEM((_CC,), i32),
            pltpu.VMEM((_CC,), i32),
            pltpu.VMEM((_CC, _H), f32),
            pltpu.SemaphoreType.DMA,
            pltpu.SemaphoreType.DMA,
        ],
    )(x, pos1, pos2)

    shared_out = pl.pallas_call(
        _shared_body,
        grid=(_T // _TB,),
        in_specs=[
            pl.BlockSpec((_TB, _H), lambda t: (t, 0)),
            pl.BlockSpec(shared_gate_w.shape, lambda t: (0, 0)),
            pl.BlockSpec(shared_up_w.shape, lambda t: (0, 0)),
            pl.BlockSpec(shared_down_w.shape, lambda t: (0, 0)),
        ],
        out_specs=pl.BlockSpec((_TB, _H), lambda t: (t, 0)),
        out_shape=jax.ShapeDtypeStruct((_T, _H), f32),
        compiler_params=pltpu.CompilerParams(
            dimension_semantics=("arbitrary",)),
    )(x, shared_gate_w, shared_up_w, shared_down_w)

    grid_spec = pltpu.PrefetchScalarGridSpec(
        num_scalar_prefetch=2,
        grid=(_NB,),
        in_specs=[
            pl.BlockSpec((_BM, _H), lambda b, be, na: (b, 0)),
            pl.BlockSpec((1, _I, _H), lambda b, be, na: (be[0, b], 0, 0)),
            pl.BlockSpec((1, _I, _H), lambda b, be, na: (be[0, b], 0, 0)),
            pl.BlockSpec((1, _H, _I), lambda b, be, na: (be[0, b], 0, 0)),
        ],
        out_specs=pl.BlockSpec((_BM, _H), lambda b, be, na: (b, 0)),
    )
    h_sorted = pl.pallas_call(
        _gmm_body,
        grid_spec=grid_spec,
        out_shape=jax.ShapeDtypeStruct((_NP, _H), f32),
        compiler_params=pltpu.CompilerParams(
            dimension_semantics=("arbitrary",)),
    )(bexp, nact, xs, gate_w, up_w, down_w)

    h1, h2 = pl.kernel(
        _sc_unsort_body, mesh=mesh,
        out_type=[
            jax.ShapeDtypeStruct((_T, _H), f32),
            jax.ShapeDtypeStruct((_T, _H), f32),
        ],
        scratch_types=[
            pltpu.VMEM((_CC,), i32),
            pltpu.VMEM((_CC, _H), f32),
            pltpu.SemaphoreType.DMA,
        ],
    )(h_sorted, pos1, pos2)

    out = pl.pallas_call(
        _add_body,
        grid=(_T // _AB,),
        in_specs=[pl.BlockSpec((_AB, _H), lambda t: (t, 0))] * 3
        + [pl.BlockSpec((_AB, 2), lambda t: (t, 0))],
        out_specs=pl.BlockSpec((_AB, _H), lambda t: (t, 0)),
        out_shape=jax.ShapeDtypeStruct((_T, _H), f32),
        compiler_params=pltpu.CompilerParams(
            dimension_semantics=("arbitrary",)),
    )(shared_out, h1, h2, wts)

    return out.reshape(bsz, seq, hdim)


# speedup vs baseline: 1.2354x; 1.0572x over previous
import jax
import jax.numpy as jnp
from jax import lax
from jax.experimental import pallas as pl
from jax.experimental.pallas import tpu as pltpu
from jax.experimental.pallas import tpu_sc as plsc

_E = 8
_I = 512
_T = 2048
_H = 1024
_TB = 512
_RB = 512
_AB = 1024
_BM = 512
_N = _T * 2
_NP = _N + _E * _BM
_NB = _NP // _BM
_NW = 32
_CC = _T // _NW


def _silu(x):
    return x * jax.nn.sigmoid(x)


def _dot(a, b, dims):
    return jax.lax.dot_general(a, b, (dims, ((), ())),
                               preferred_element_type=jnp.float32,
                               precision=jax.lax.Precision.DEFAULT)


def _router_body(x_ref, wg_ref, wts_ref, pos1_ref, pos2_ref, bexp_ref,
                 nact_ref, sel_s, wts_s, counts_s, poff_s, carry_s):
    p = pl.program_id(0)
    t = pl.program_id(1)
    hp = jax.lax.Precision.HIGHEST

    @pl.when(p == 0)
    def _pass0():
        x = x_ref[...]
        logits = _dot(x, wg_ref[...], ((1,), (1,)))
        m = jnp.max(logits, axis=1, keepdims=True)
        pr = jnp.exp(logits - m)
        iota = jax.lax.broadcasted_iota(jnp.int32, pr.shape, 1)
        m1 = jnp.max(pr, axis=1, keepdims=True)
        i1 = jnp.min(jnp.where(pr == m1, iota, _E), axis=1, keepdims=True)
        pr2 = jnp.where(iota == i1, -1.0, pr)
        m2 = jnp.max(pr2, axis=1, keepdims=True)
        i2 = jnp.min(jnp.where(pr2 == m2, iota, _E), axis=1, keepdims=True)
        den = m1 + m2 + 1e-20
        w12 = jnp.concatenate([m1 / den, m2 / den], axis=1)
        wts_ref[...] = w12
        sel_s[pl.ds(t * _RB, _RB), :] = jnp.concatenate([i1, i2], axis=1)
        wts_s[pl.ds(t * _RB, _RB), :] = w12

        both = ((iota == i1) | (iota == i2)).astype(jnp.float32)
        bsum = jnp.sum(both, axis=0, keepdims=True)

        @pl.when(t == 0)
        def _():
            counts_s[...] = bsum

        @pl.when(t > 0)
        def _():
            counts_s[...] += bsum

        @pl.when(t == (_T // _RB) - 1)
        def _finalize():
            counts = counts_s[...]
            padded = jnp.floor((counts + (_BM - 1)) / _BM) * _BM
            ir = jax.lax.broadcasted_iota(jnp.int32, (_E, _E), 0)
            ic = jax.lax.broadcasted_iota(jnp.int32, (_E, _E), 1)
            tri = (ir <= ic).astype(jnp.float32)
            cpad = jax.lax.dot_general(padded, tri, (((1,), (0,)), ((), ())),
                                       preferred_element_type=jnp.float32,
                                       precision=hp)
            poff_s[...] = cpad - padded
            nact_ref[...] = (cpad[:, _E - 1:] / _BM).astype(jnp.int32)
            cpadB = jnp.broadcast_to(cpad, (_NB, _E))
            bsB = (jax.lax.broadcasted_iota(jnp.int32, (_NB, _E), 0)
                   .astype(jnp.float32) * _BM)
            be = jnp.sum((cpadB <= bsB).astype(jnp.int32), axis=1)
            bexp_ref[...] = jnp.minimum(be, _E - 1)[None, :]

    @pl.when(p == 1)
    def _pass1():
        sel = sel_s[pl.ds(t * _RB, _RB), :]
        iota = jax.lax.broadcasted_iota(jnp.int32, (_RB, _E), 1)
        oh1 = (iota == sel[:, 0][:, None]).astype(jnp.float32)
        oh2 = (iota == sel[:, 1][:, None]).astype(jnp.float32)
        both = oh1 + oh2

        @pl.when(t == 0)
        def _():
            carry_s[...] = jnp.zeros_like(carry_s)

        ir = jax.lax.broadcasted_iota(jnp.int32, (_RB, _RB), 0)
        ic = jax.lax.broadcasted_iota(jnp.int32, (_RB, _RB), 1)
        lstrict = (ir > ic).astype(jnp.float32)
        cume = jax.lax.dot_general(lstrict, both, (((1,), (0,)), ((), ())),
                                   preferred_element_type=jnp.float32,
                                   precision=hp)
        base = poff_s[...] + carry_s[...] + cume
        pos1_ref[...] = jnp.sum(oh1 * base, axis=1).astype(jnp.int32)
        pos2_ref[...] = jnp.sum(oh2 * base, axis=1).astype(jnp.int32)
        carry_s[...] += jnp.sum(both, axis=0, keepdims=True)
        wts_ref[...] = wts_s[pl.ds(t * _RB, _RB), :]


def _shared_body(x_ref, sgw_ref, suw_ref, sdw_ref, out_ref):
    x = x_ref[...]
    g = _dot(x, sgw_ref[...], ((1,), (1,)))
    u = _dot(x, suw_ref[...], ((1,), (1,)))
    out_ref[...] = _dot(_silu(g) * u, sdw_ref[...], ((1,), (1,)))


def _gmm_body(bexp_ref, nact_ref, xs_ref, gw_ref, uw_ref, dw_ref, h_ref):
    b = pl.program_id(0)

    @pl.when(b < nact_ref[0, 0])
    def _compute():
        x = xs_ref[...]
        g = _dot(x, gw_ref[0], ((1,), (1,)))
        u = _dot(x, uw_ref[0], ((1,), (1,)))
        a = _silu(g) * u
        h_ref[...] = _dot(a, dw_ref[0], ((1,), (1,)))


def _sc_dispatch_body(x_hbm, pos1_hbm, pos2_hbm, xs_hbm,
                      p1_v, p2_v, rows_v, sem1, sem2):
    wid = lax.axis_index("s") * 2 + lax.axis_index("c")
    tb = wid * _CC
    pltpu.sync_copy(pos1_hbm.at[pl.ds(tb, _CC)], p1_v)
    pltpu.sync_copy(pos2_hbm.at[pl.ds(tb, _CC)], p2_v)
    pltpu.sync_copy(x_hbm.at[pl.ds(tb, _CC)], rows_v)
    c1 = pltpu.async_copy(rows_v, xs_hbm.at[p1_v], sem1)
    c2 = pltpu.async_copy(rows_v, xs_hbm.at[p2_v], sem2)
    c1.wait()
    c2.wait()


def _sc_unsort_body(h_hbm, pos1_hbm, pos2_hbm, h1_hbm, h2_hbm,
                    idx_v, rows_v, sem):
    wid = lax.axis_index("s") * 2 + lax.axis_index("c")
    gb = wid * _CC
    for pos_hbm, dst_hbm in ((pos1_hbm, h1_hbm), (pos2_hbm, h2_hbm)):
        pltpu.sync_copy(pos_hbm.at[pl.ds(gb, _CC)], idx_v)
        pltpu.async_copy(h_hbm.at[idx_v], rows_v, sem).wait()
        pltpu.sync_copy(rows_v, dst_hbm.at[pl.ds(gb, _CC)])


def _add_body(s_ref, h1_ref, h2_ref, w_ref, out_ref):
    w = w_ref[...]
    out_ref[...] = (s_ref[...] + w[:, 0][:, None] * h1_ref[...]
                    + w[:, 1][:, None] * h2_ref[...])


def kernel(hidden_states, W_gate, gate_w, up_w, down_w,
           shared_gate_w, shared_up_w, shared_down_w):
    bsz, seq, hdim = hidden_states.shape
    x = hidden_states.reshape(_T, _H)
    f32 = jnp.float32
    i32 = jnp.int32

    wts, pos1, pos2, bexp, nact = pl.pallas_call(
        _router_body,
        grid=(2, _T // _RB),
        in_specs=[
            pl.BlockSpec((_RB, _H), lambda p, t: (t * (1 - p), 0)),
            pl.BlockSpec((_E, _H), lambda p, t: (0, 0)),
        ],
        out_specs=[
            pl.BlockSpec((_RB, 2), lambda p, t: (t, 0)),
            pl.BlockSpec((_RB,), lambda p, t: (t,)),
            pl.BlockSpec((_RB,), lambda p, t: (t,)),
            pl.BlockSpec((1, _NB), lambda p, t: (0, 0)),
            pl.BlockSpec((1, 1), lambda p, t: (0, 0)),
        ],
        out_shape=[
            jax.ShapeDtypeStruct((_T, 2), f32),
            jax.ShapeDtypeStruct((_T,), i32),
            jax.ShapeDtypeStruct((_T,), i32),
            jax.ShapeDtypeStruct((1, _NB), i32),
            jax.ShapeDtypeStruct((1, 1), i32),
        ],
        scratch_shapes=[
            pltpu.VMEM((_T, 2), i32),
            pltpu.VMEM((_T, 2), f32),
            pltpu.VMEM((1, _E), f32),
            pltpu.VMEM((1, _E), f32),
            pltpu.VMEM((1, _E), f32),
        ],
        compiler_params=pltpu.CompilerParams(
            dimension_semantics=("arbitrary", "arbitrary")),
    )(x, W_gate)

    mesh = plsc.VectorSubcoreMesh(core_axis_name="c", subcore_axis_name="s")

    xs = pl.kernel(
        _sc_dispatch_body, mesh=mesh,
        out_type=jax.ShapeDtypeStruct((_NP, _H), f32),
        scratch_types=[
            pltpu.VMEM((_CC,), i32),
            pltpu.VMEM((_CC,), i32),
            pltpu.VMEM((_CC, _H), f32),
            pltpu.SemaphoreType.DMA,
            pltpu.SemaphoreType.DMA,
        ],
    )(x, pos1, pos2)

    shared_out = pl.pallas_call(
        _shared_body,
        grid=(_T // _TB,),
        in_specs=[
            pl.BlockSpec((_TB, _H), lambda t: (t, 0)),
            pl.BlockSpec(shared_gate_w.shape, lambda t: (0, 0)),
            pl.BlockSpec(shared_up_w.shape, lambda t: (0, 0)),
            pl.BlockSpec(shared_down_w.shape, lambda t: (0, 0)),
        ],
        out_specs=pl.BlockSpec((_TB, _H), lambda t: (t, 0)),
        out_shape=jax.ShapeDtypeStruct((_T, _H), f32),
        compiler_params=pltpu.CompilerParams(
            dimension_semantics=("arbitrary",)),
    )(x, shared_gate_w, shared_up_w, shared_down_w)

    grid_spec = pltpu.PrefetchScalarGridSpec(
        num_scalar_prefetch=2,
        grid=(_NB,),
        in_specs=[
            pl.BlockSpec(
                (_BM, _H),
                lambda b, be, na: (jnp.minimum(b, na[0, 0] - 1), 0)),
            pl.BlockSpec(
                (1, _I, _H),
                lambda b, be, na: (be[0, jnp.minimum(b, na[0, 0] - 1)], 0, 0)),
            pl.BlockSpec(
                (1, _I, _H),
                lambda b, be, na: (be[0, jnp.minimum(b, na[0, 0] - 1)], 0, 0)),
            pl.BlockSpec(
                (1, _H, _I),
                lambda b, be, na: (be[0, jnp.minimum(b, na[0, 0] - 1)], 0, 0)),
        ],
        out_specs=pl.BlockSpec(
            (_BM, _H), lambda b, be, na: (jnp.minimum(b, na[0, 0] - 1), 0)),
    )
    h_sorted = pl.pallas_call(
        _gmm_body,
        grid_spec=grid_spec,
        out_shape=jax.ShapeDtypeStruct((_NP, _H), f32),
        compiler_params=pltpu.CompilerParams(
            dimension_semantics=("arbitrary",)),
    )(bexp, nact, xs, gate_w, up_w, down_w)

    h1, h2 = pl.kernel(
        _sc_unsort_body, mesh=mesh,
        out_type=[
            jax.ShapeDtypeStruct((_T, _H), f32),
            jax.ShapeDtypeStruct((_T, _H), f32),
        ],
        scratch_types=[
            pltpu.VMEM((_CC,), i32),
            pltpu.VMEM((_CC, _H), f32),
            pltpu.SemaphoreType.DMA,
        ],
    )(h_sorted, pos1, pos2)

    out = pl.pallas_call(
        _add_body,
        grid=(_T // _AB,),
        in_specs=[pl.BlockSpec((_AB, _H), lambda t: (t, 0))] * 3
        + [pl.BlockSpec((_AB, 2), lambda t: (t, 0))],
        out_specs=pl.BlockSpec((_AB, _H), lambda t: (t, 0)),
        out_shape=jax.ShapeDtypeStruct((_T, _H), f32),
        compiler_params=pltpu.CompilerParams(
            dimension_semantics=("arbitrary",)),
    )(shared_out, h1, h2, wts)

    return out.reshape(bsz, seq, hdim)
